# baseline (speedup 1.0000x reference)
import functools

import jax
import jax.numpy as jnp
from jax import lax
from jax.experimental import pallas as pl
from jax.experimental.pallas import tpu as pltpu
from jax.experimental.pallas import tpu_sc as plsc

D = 1024
BATCH = 4
SEQ = 2048
N_TOK = BATCH * SEQ
NC = 2
NS = 16
NW = NC * NS
CHUNK = 32
NBUF = 3

SLAB_BATCH = 2
SLAB_TOK = SLAB_BATCH * SEQ
TOK_PER_W = SLAB_TOK // NW
N_STEP = TOK_PER_W // CHUNK
W_PER_ROW = SEQ // TOK_PER_W

_mesh = plsc.VectorSubcoreMesh(
    core_axis_name="c", subcore_axis_name="s", num_cores=NC, num_subcores=NS
)


@functools.partial(
    pl.kernel,
    out_type=jax.ShapeDtypeStruct((SLAB_TOK, D), jnp.float32),
    mesh=_mesh,
    scratch_types=[
        pltpu.VMEM((NBUF, CHUNK), jnp.int32),
        pltpu.VMEM((NBUF, CHUNK, D), jnp.float32),
        pltpu.SemaphoreType.DMA((NBUF,)),
        pltpu.SemaphoreType.DMA((NBUF,)),
    ],
)
def _gather_kernel(ids_hbm, tok_hbm, out_hbm, idx_v, rows_v, sem_g, sem_s):
    wid = lax.axis_index("s") * NC + lax.axis_index("c")
    row = wid // W_PER_ROW
    col0 = (wid % W_PER_ROW) * TOK_PER_W
    base = wid * TOK_PER_W

    def start_gather(step, nb):
        pltpu.sync_copy(ids_hbm.at[row, pl.ds(col0 + step * CHUNK, CHUNK)],
                        idx_v.at[nb])
        pltpu.async_copy(tok_hbm.at[idx_v.at[nb]], rows_v.at[nb], sem_g.at[nb])

    def wait_store(nb):
        pltpu.make_async_copy(
            rows_v.at[nb], out_hbm.at[pl.ds(0, CHUNK)], sem_s.at[nb]
        ).wait()

    for ph in range(N_STEP + 2):
        if ph < N_STEP:
            if ph >= NBUF:
                wait_store(ph % NBUF)
            start_gather(ph, ph % NBUF)
        if ph >= 2:
            s = ph - 2
            nb = s % NBUF
            pltpu.make_async_copy(
                tok_hbm.at[idx_v.at[nb]], rows_v.at[nb], sem_g.at[nb]
            ).wait()
            pltpu.async_copy(
                rows_v.at[nb], out_hbm.at[pl.ds(base + s * CHUNK, CHUNK)],
                sem_s.at[nb],
            )
    for nb in range(min(NBUF, N_STEP)):
        wait_store(nb)


TC_BLK = 1024
SLAB_BLKS = SLAB_TOK // TC_BLK
P_BLKS = SEQ // TC_BLK


def _ln_body(emb_ref, pos_ref, gam_ref, bet_ref, out_ref):
    x = emb_ref[...] + pos_ref[...]
    m = jnp.mean(x, axis=-1, keepdims=True)
    xc = x - m
    v = jnp.mean(xc * xc, axis=-1, keepdims=True)
    out_ref[...] = xc * lax.rsqrt(v + 1e-5) * gam_ref[...] + bet_ref[...]


def _ln_body_alias(emb_ref, pos_ref, gam_ref, bet_ref, prev_ref, out_ref):
    _ln_body(emb_ref, pos_ref, gam_ref, bet_ref, out_ref)


def _ln_specs(slab):
    return dict(
        grid=(P_BLKS, SLAB_BATCH),
        in_specs=[
            pl.BlockSpec((TC_BLK, D), lambda p, b: (b * P_BLKS + p, 0)),
            pl.BlockSpec((TC_BLK, D), lambda p, b: (p, 0)),
            pl.BlockSpec((1, D), lambda p, b: (0, 0)),
            pl.BlockSpec((1, D), lambda p, b: (0, 0)),
        ],
        out_specs=pl.BlockSpec(
            (TC_BLK, D),
            lambda p, b, slab=slab: (slab * SLAB_BLKS + b * P_BLKS + p, 0),
        ),
    )


_spec_a = _ln_specs(0)
_ln_call_a = pl.pallas_call(
    _ln_body,
    out_shape=jax.ShapeDtypeStruct((N_TOK, D), jnp.float32),
    **_spec_a,
)

_spec_b = _ln_specs(1)
_spec_b["in_specs"] = _spec_b["in_specs"] + [
    pl.BlockSpec(memory_space=pl.ANY)
]
_ln_call_b = pl.pallas_call(
    _ln_body_alias,
    out_shape=jax.ShapeDtypeStruct((N_TOK, D), jnp.float32),
    input_output_aliases={4: 0},
    **_spec_b,
)


def kernel(input_ids, token_table, pos_table, ln_gamma, ln_beta):
    ids = input_ids
    if ids.dtype != jnp.int32:
        ids = ids.astype(jnp.int32)
    gam = ln_gamma.reshape(1, D)
    bet = ln_beta.reshape(1, D)
    emb_a = _gather_kernel(ids[:SLAB_BATCH], token_table)
    emb_b = _gather_kernel(ids[SLAB_BATCH:], token_table)
    out = _ln_call_a(emb_a, pos_table, gam, bet)
    out = _ln_call_b(emb_b, pos_table, gam, bet, out)
    return out.reshape(BATCH, SEQ, D)


# speedup vs baseline: 1.0464x; 1.0464x over previous
import functools

import jax
import jax.numpy as jnp
from jax import lax
from jax.experimental import pallas as pl
from jax.experimental.pallas import tpu as pltpu
from jax.experimental.pallas import tpu_sc as plsc

D = 1024
BATCH = 4
SEQ = 2048
N_TOK = BATCH * SEQ
NC = 2
NS = 16
NW = NC * NS
CHUNK = 32
TOK_PER_W = N_TOK // NW
N_STEP = TOK_PER_W // CHUNK

_mesh = plsc.VectorSubcoreMesh(
    core_axis_name="c", subcore_axis_name="s", num_cores=NC, num_subcores=NS
)


NBUF = 3


@functools.partial(
    pl.kernel,
    out_type=jax.ShapeDtypeStruct((N_TOK, D), jnp.float32),
    mesh=_mesh,
    scratch_types=[
        pltpu.VMEM((TOK_PER_W,), jnp.int32),
        pltpu.VMEM((NBUF, CHUNK, D), jnp.float32),
        pltpu.SemaphoreType.DMA((NBUF,)),
        pltpu.SemaphoreType.DMA((NBUF,)),
    ],
)
def _gather_kernel(ids_hbm, tok_hbm, out_hbm, idx_v, rows_v, sem_g, sem_s):
    wid = lax.axis_index("s") * NC + lax.axis_index("c")
    row = wid // (SEQ // TOK_PER_W)
    col0 = (wid % (SEQ // TOK_PER_W)) * TOK_PER_W
    base = wid * TOK_PER_W

    pltpu.sync_copy(ids_hbm.at[row, pl.ds(col0, TOK_PER_W)], idx_v)

    def start_gather(step, nb):
        pltpu.async_copy(tok_hbm.at[idx_v.at[pl.ds(step * CHUNK, CHUNK)]], rows_v.at[nb], sem_g.at[nb])

    def wait_store(nb):
        pltpu.make_async_copy(
            rows_v.at[nb], out_hbm.at[pl.ds(0, CHUNK)], sem_s.at[nb]
        ).wait()

    def wait_gather_start_store(step, nb):
        pltpu.make_async_copy(
            tok_hbm.at[idx_v.at[pl.ds(step * CHUNK, CHUNK)]], rows_v.at[nb], sem_g.at[nb]
        ).wait()
        pltpu.async_copy(
            rows_v.at[nb], out_hbm.at[pl.ds(base + step * CHUNK, CHUNK)],
            sem_s.at[nb],
        )

    start_gather(0, 0)
    start_gather(1, 1)

    def tri_body(i, carry):
        for k in range(NBUF):
            step = NBUF * i + k
            kk = (k + 2) % NBUF
            @pl.when(step >= 1)
            def _():
                wait_store(kk)
            start_gather(step + 2, kk)
            wait_gather_start_store(step, k)
        return carry

    lax.fori_loop(0, (N_STEP - 2) // NBUF, tri_body, 0)

    for s in range(N_STEP - 2, N_STEP):
        wait_gather_start_store(s, s % NBUF)
    for nb in range(NBUF):
        wait_store(nb)


TC_BLK = 2048


def _ln_body(emb_ref, pos_ref, gam_ref, bet_ref, out_ref):
    x = emb_ref[...] + pos_ref[...]
    m = jnp.mean(x, axis=-1, keepdims=True)
    xc = x - m
    v = jnp.mean(xc * xc, axis=-1, keepdims=True)
    out_ref[...] = xc * lax.rsqrt(v + 1e-5) * gam_ref[...] + bet_ref[...]


_ln_call = pl.pallas_call(
    _ln_body,
    out_shape=jax.ShapeDtypeStruct((N_TOK, D), jnp.float32),
    grid=(SEQ // TC_BLK, BATCH),
    in_specs=[
        pl.BlockSpec((TC_BLK, D), lambda p, b: (b * (SEQ // TC_BLK) + p, 0)),
        pl.BlockSpec((TC_BLK, D), lambda p, b: (p, 0)),
        pl.BlockSpec((1, D), lambda p, b: (0, 0)),
        pl.BlockSpec((1, D), lambda p, b: (0, 0)),
    ],
    out_specs=pl.BlockSpec((TC_BLK, D), lambda p, b: (b * (SEQ // TC_BLK) + p, 0)),
)


def kernel(input_ids, token_table, pos_table, ln_gamma, ln_beta):
    ids = input_ids
    if ids.dtype != jnp.int32:
        ids = ids.astype(jnp.int32)
    emb = _gather_kernel(ids, token_table)
    out = _ln_call(emb, pos_table, ln_gamma.reshape(1, D), ln_beta.reshape(1, D))
    return out.reshape(BATCH, SEQ, D)


# speedup vs baseline: 1.0490x; 1.0025x over previous
import functools

import jax
import jax.numpy as jnp
from jax import lax
from jax.experimental import pallas as pl
from jax.experimental.pallas import tpu as pltpu
from jax.experimental.pallas import tpu_sc as plsc

D = 1024
BATCH = 4
SEQ = 2048
N_TOK = BATCH * SEQ
NC = 2
NS = 16
NW = NC * NS
CHUNK = 16
TOK_PER_W = N_TOK // NW
N_STEP = TOK_PER_W // CHUNK

_mesh = plsc.VectorSubcoreMesh(
    core_axis_name="c", subcore_axis_name="s", num_cores=NC, num_subcores=NS
)


NBUF = 6
DEPTH = 4


@functools.partial(
    pl.kernel,
    out_type=jax.ShapeDtypeStruct((N_TOK, D), jnp.float32),
    mesh=_mesh,
    scratch_types=[
        pltpu.VMEM((TOK_PER_W,), jnp.int32),
        pltpu.VMEM((NBUF, CHUNK, D), jnp.float32),
        pltpu.SemaphoreType.DMA((NBUF,)),
        pltpu.SemaphoreType.DMA((NBUF,)),
    ],
)
def _gather_kernel(ids_hbm, tok_hbm, out_hbm, idx_v, rows_v, sem_g, sem_s):
    wid = lax.axis_index("s") * NC + lax.axis_index("c")
    row = wid // (SEQ // TOK_PER_W)
    col0 = (wid % (SEQ // TOK_PER_W)) * TOK_PER_W
    base = wid * TOK_PER_W

    pltpu.sync_copy(ids_hbm.at[row, pl.ds(col0, TOK_PER_W)], idx_v)

    def start_gather(step, nb):
        pltpu.async_copy(tok_hbm.at[idx_v.at[pl.ds(step * CHUNK, CHUNK)]], rows_v.at[nb], sem_g.at[nb])

    def wait_store(nb):
        pltpu.make_async_copy(
            rows_v.at[nb], out_hbm.at[pl.ds(0, CHUNK)], sem_s.at[nb]
        ).wait()

    def wait_gather_start_store(step, nb):
        pltpu.make_async_copy(
            tok_hbm.at[idx_v.at[pl.ds(step * CHUNK, CHUNK)]], rows_v.at[nb], sem_g.at[nb]
        ).wait()
        pltpu.async_copy(
            rows_v.at[nb], out_hbm.at[pl.ds(base + step * CHUNK, CHUNK)],
            sem_s.at[nb],
        )

    for ph in range(N_STEP + DEPTH):
        if ph < N_STEP:
            if ph >= NBUF:
                wait_store(ph % NBUF)
            start_gather(ph, ph % NBUF)
        if ph >= DEPTH:
            wait_gather_start_store(ph - DEPTH, (ph - DEPTH) % NBUF)
    for nb in range(NBUF):
        wait_store(nb)


TC_BLK = 2048


def _ln_body(emb_ref, pos_ref, gam_ref, bet_ref, out_ref):
    x = emb_ref[...] + pos_ref[...]
    m = jnp.mean(x, axis=-1, keepdims=True)
    xc = x - m
    v = jnp.mean(xc * xc, axis=-1, keepdims=True)
    out_ref[...] = xc * lax.rsqrt(v + 1e-5) * gam_ref[...] + bet_ref[...]


_ln_call = pl.pallas_call(
    _ln_body,
    out_shape=jax.ShapeDtypeStruct((N_TOK, D), jnp.float32),
    grid=(SEQ // TC_BLK, BATCH),
    in_specs=[
        pl.BlockSpec((TC_BLK, D), lambda p, b: (b * (SEQ // TC_BLK) + p, 0)),
        pl.BlockSpec((TC_BLK, D), lambda p, b: (p, 0)),
        pl.BlockSpec((1, D), lambda p, b: (0, 0)),
        pl.BlockSpec((1, D), lambda p, b: (0, 0)),
    ],
    out_specs=pl.BlockSpec((TC_BLK, D), lambda p, b: (b * (SEQ // TC_BLK) + p, 0)),
)


def kernel(input_ids, token_table, pos_table, ln_gamma, ln_beta):
    ids = input_ids
    if ids.dtype != jnp.int32:
        ids = ids.astype(jnp.int32)
    emb = _gather_kernel(ids, token_table)
    out = _ln_call(emb, pos_table, ln_gamma.reshape(1, D), ln_beta.reshape(1, D))
    return out.reshape(BATCH, SEQ, D)
